# baseline (speedup 1.0000x reference)
import functools

import numpy as np
import jax
import jax.numpy as jnp
from jax import lax
from jax.experimental import pallas as pl
from jax.experimental.pallas import tpu as pltpu

_LANE = 128
_WIN = 400
_STRIDE = 160
_HALF = _STRIDE // 2
_FFT_LEN = 512
_F = _FFT_LEN // 2 + 1
_F_SPLIT = 264
_C = 2 * _F_SPLIT
_PAD = _WIN - _STRIDE
_TERMS = ((-2, 1), (-1, 0), (-1, 1), (0, 0), (0, 1))


def _round_up(x, m):
    return ((x + m - 1) // m) * m


def _build_weights():
    n = np.arange(_WIN)
    window = 0.54 - 0.46 * np.cos(2.0 * np.pi * n / _WIN)
    basis = np.fft.rfft(np.eye(_FFT_LEN))[:_WIN]
    kern = np.concatenate([np.real(basis), np.imag(basis)], 1).T * window
    w = np.zeros((_C, _WIN), np.float32)
    w[:_F] = kern[:_F]
    w[_F_SPLIT:_F_SPLIT + _F] = kern[_F:]
    return np.stack([w[:, _HALF * i:_HALF * (i + 1)]
                     for i in range(len(_TERMS))])


def _atan2_poly(y, x):
    ax = jnp.abs(x)
    ay = jnp.abs(y)
    hi = jnp.maximum(ax, ay)
    lo = jnp.minimum(ax, ay)
    t = lo / jnp.maximum(hi, 1e-30)
    t2 = t * t
    p = 0.0208351
    p = p * t2 - 0.0851330
    p = p * t2 + 0.1801410
    p = p * t2 - 0.3302995
    p = p * t2 + 0.9998660
    a = p * t
    a = jnp.where(ay > ax, (0.5 * np.pi) - a, a)
    a = jnp.where(x < 0.0, np.pi - a, a)
    return jnp.where(y < 0.0, -a, a)


def _stft_kernel(x_ref, w_ref, mags_ref, phase_ref, *, tile_t, n_xrows):
    blk = x_ref[...]
    zeros_front = jnp.zeros((2, _STRIDE), jnp.float32)
    zeros_back = jnp.zeros((tile_t - n_xrows, _STRIDE), jnp.float32)
    rows = jnp.concatenate([zeros_front, blk, zeros_back], axis=0)
    acc = None
    for i, (d, h) in enumerate(_TERMS):
        xs = rows[d + 2:d + 2 + tile_t, h * _HALF:(h + 1) * _HALF]
        part = lax.dot_general(w_ref[i], xs, (((1,), (1,)), ((), ())),
                               preferred_element_type=jnp.float32)
        acc = part if acc is None else acc + part
    real = acc[:_F_SPLIT, :]
    imag = acc[_F_SPLIT:, :]
    r2 = real * real + imag * imag
    mags = r2 * lax.rsqrt(r2 + 1e-30)
    ph = _atan2_poly(imag, real)
    mags_ref[...] = mags[:_F, :]
    phase_ref[...] = ph[:_F, :]


def kernel(inputs):
    if inputs.ndim == 3:
        inputs = inputs.reshape(inputs.shape[0], inputs.shape[-1])
    x = inputs.astype(jnp.float32)
    T_out = (x.shape[1] + 2 * _PAD - _WIN) // _STRIDE + 1
    if x.shape[1] % _STRIDE:
        x = jnp.pad(x, ((0, 0), (0, _STRIDE - x.shape[1] % _STRIDE)))
    B, T = x.shape
    n_xrows = T // _STRIDE
    tile_t = _round_up(T_out, _LANE)

    sig = x.reshape(B, n_xrows, _STRIDE)
    w = jnp.asarray(_build_weights())

    out_spec = pl.BlockSpec((None, _F, tile_t), lambda b: (b, 0, 0))
    mags, phase = pl.pallas_call(
        functools.partial(_stft_kernel, tile_t=tile_t, n_xrows=n_xrows),
        out_shape=(jax.ShapeDtypeStruct((B, _F, T_out), jnp.float32),
                   jax.ShapeDtypeStruct((B, _F, T_out), jnp.float32)),
        grid=(B,),
        in_specs=[
            pl.BlockSpec((None, n_xrows, _STRIDE), lambda b: (b, 0, 0)),
            pl.BlockSpec((len(_TERMS), _C, _HALF), lambda b: (0, 0, 0)),
        ],
        out_specs=(out_spec, out_spec),
        compiler_params=pltpu.CompilerParams(
            dimension_semantics=("parallel",)),
    )(sig, w)
    return mags, phase
